# Initial kernel scaffold; baseline (speedup 1.0000x reference)
import jax
import jax.numpy as jnp
from jax.experimental import pallas as pl


def kernel(inputs, table, W, b):
    raise NotImplementedError("write your pallas kernel here")


# speedup vs baseline: 13.1188x; 13.1188x over previous
import jax
import jax.numpy as jnp
from jax import lax
from jax.experimental import pallas as pl
from jax.experimental.pallas import tpu as pltpu
from jax.experimental.pallas import tpu_sc as plsc

_NC = 2
_NS = 16
_NW = _NC * _NS
_L = 200
_B = 4096
_EMB = 128
_LBL = 100
_EPW = _B // _NW
_HALF = _L // 2
_VEC = 16
_KV = _EMB // _VEC
_OUT_TILE = 16
_SCALE = 1.0 / _L


def _sc_pool_body(idx_hbm, table_hbm, out_hbm, idx_v, rows0, rows1, outst,
                  sem0, sem1):
    wid = lax.axis_index("s") * _NC + lax.axis_index("c")
    base = pl.multiple_of(wid * _EPW, _EPW)
    pltpu.sync_copy(idx_hbm.at[pl.ds(base * 2, 2 * _EPW)], idx_v)

    def issue(e, buf, sem):
        row = e * 2
        pltpu.async_copy(table_hbm.at[idx_v.at[row]], buf.at[0], sem)
        pltpu.async_copy(table_hbm.at[idx_v.at[row + 1]], buf.at[1], sem)

    def drain(buf, sem):
        for r in range(2):
            pltpu.make_async_copy(table_hbm.at[idx_v.at[0]], buf.at[r],
                                  sem).wait()

    def consume(e, buf):
        def body(l, acc):
            return tuple(
                acc[k]
                + buf[0, l, pl.ds(k * _VEC, _VEC)]
                + buf[1, l, pl.ds(k * _VEC, _VEC)]
                for k in range(_KV))

        acc = lax.fori_loop(
            0, _HALF, body,
            tuple(jnp.zeros((_VEC,), jnp.float32) for _ in range(_KV)))
        m = lax.rem(e, _OUT_TILE)
        for k in range(_KV):
            outst[m, pl.ds(k * _VEC, _VEC)] = acc[k] * _SCALE

        @pl.when(m == _OUT_TILE - 1)
        def _flush():
            start = pl.multiple_of(base + e - (_OUT_TILE - 1), _OUT_TILE)
            pltpu.sync_copy(outst, out_hbm.at[pl.ds(start, _OUT_TILE)])

    issue(0, rows0, sem0)

    def step(g, carry):
        e0 = 2 * g
        issue(e0 + 1, rows1, sem1)
        drain(rows0, sem0)
        consume(e0, rows0)

        @pl.when(e0 + 2 < _EPW)
        def _():
            issue(e0 + 2, rows0, sem0)

        drain(rows1, sem1)
        consume(e0 + 1, rows1)
        return carry

    lax.fori_loop(0, _EPW // 2, step, jnp.int32(0))


def _head_body(z_ref, w_ref, b_ref, o_ref):
    o_ref[...] = jax.nn.sigmoid(
        jnp.dot(z_ref[...], w_ref[...], preferred_element_type=jnp.float32)
        + b_ref[...])


def kernel(inputs, table, W, b):
    idx2d = jnp.transpose(inputs).reshape(_B * 2, _HALF)
    mesh = plsc.VectorSubcoreMesh(core_axis_name="c", subcore_axis_name="s")
    pooled = pl.kernel(
        _sc_pool_body,
        mesh=mesh,
        out_type=jax.ShapeDtypeStruct((_B, _EMB), jnp.float32),
        scratch_types=[
            pltpu.VMEM((2 * _EPW, _HALF), jnp.int32),
            pltpu.VMEM((2, _HALF, _EMB), jnp.float32),
            pltpu.VMEM((2, _HALF, _EMB), jnp.float32),
            pltpu.VMEM((_OUT_TILE, _EMB), jnp.float32),
            pltpu.SemaphoreType.DMA,
            pltpu.SemaphoreType.DMA,
        ],
    )(idx2d, table)
    out = pl.pallas_call(
        _head_body,
        grid=(4,),
        in_specs=[
            pl.BlockSpec((_B // 4, _EMB), lambda i: (i, 0)),
            pl.BlockSpec((_EMB, _LBL), lambda i: (0, 0)),
            pl.BlockSpec((1, _LBL), lambda i: (0, 0)),
        ],
        out_specs=pl.BlockSpec((_B // 4, _LBL), lambda i: (i, 0)),
        out_shape=jax.ShapeDtypeStruct((_B, _LBL), jnp.float32),
    )(pooled, W, jnp.reshape(b, (1, _LBL)))
    return out


# speedup vs baseline: 15.3942x; 1.1734x over previous
import jax
import jax.numpy as jnp
from jax import lax
from jax.experimental import pallas as pl
from jax.experimental.pallas import tpu as pltpu
from jax.experimental.pallas import tpu_sc as plsc

_NC = 2
_NS = 16
_NW = _NC * _NS
_L = 200
_B = 4096
_EMB = 128
_LBL = 100
_EPW = _B // _NW
_HALF = _L // 2
_VEC = 16
_KV = _EMB // _VEC
_NBUF = 4
_IDXP = 104
_OUT_TILE = 8
_SCALE = 1.0 / _L


def _sc_pool_body(idx_hbm, table_hbm, out_hbm, idx_v,
                  rows0, rows1, rows2, rows3, outst,
                  sem0, sem1, sem2, sem3):
    bufs = (rows0, rows1, rows2, rows3)
    sems = (sem0, sem1, sem2, sem3)
    wid = lax.axis_index("s") * _NC + lax.axis_index("c")
    base = pl.multiple_of(wid * _EPW, _EPW)
    nwords = 2 * _EPW * _IDXP
    pltpu.sync_copy(
        idx_hbm.at[pl.ds(pl.multiple_of(wid * nwords, 8), nwords)], idx_v)

    def idx_ref(row):
        return idx_v.at[pl.ds(pl.multiple_of(row * _IDXP, 8), _HALF)]

    def issue(e, buf, sem):
        row = e * 2
        pltpu.async_copy(table_hbm.at[idx_ref(row)],
                         buf.at[pl.ds(0, _HALF)], sem)
        pltpu.async_copy(table_hbm.at[idx_ref(row + 1)],
                         buf.at[pl.ds(_HALF, _HALF)], sem)

    def drain(buf, sem):
        for r in range(2):
            pltpu.make_async_copy(table_hbm.at[idx_ref(0)],
                                  buf.at[pl.ds(0, _HALF)], sem).wait()

    def consume(e, buf):
        def body(l, acc):
            acc = list(acc)
            for h in range(2):
                for k in range(_KV):
                    acc[k] = acc[k] + buf[l + _HALF * h,
                                          pl.ds(k * _VEC, _VEC)]
            return tuple(acc)

        acc = lax.fori_loop(
            0, _HALF, body,
            tuple(jnp.zeros((_VEC,), jnp.float32) for _ in range(_KV)))
        m = lax.rem(e, _OUT_TILE)
        for k in range(_KV):
            outst[m, pl.ds(k * _VEC, _VEC)] = acc[k] * _SCALE

        @pl.when(m == _OUT_TILE - 1)
        def _flush():
            start = pl.multiple_of(base + e - (_OUT_TILE - 1), _OUT_TILE)
            pltpu.sync_copy(outst, out_hbm.at[pl.ds(start, _OUT_TILE)])

    for q in range(_NBUF):
        issue(q, bufs[q], sems[q])

    def step(g, carry):
        e0 = _NBUF * g
        for q in range(_NBUF):
            e = e0 + q
            drain(bufs[q], sems[q])
            consume(e, bufs[q])

            @pl.when(e + _NBUF < _EPW)
            def _():
                issue(e + _NBUF, bufs[q], sems[q])
        return carry

    lax.fori_loop(0, _EPW // _NBUF, step, jnp.int32(0))


def _head_body(z_ref, w_ref, b_ref, o_ref):
    o_ref[...] = jax.nn.sigmoid(
        jnp.dot(z_ref[...], w_ref[...], preferred_element_type=jnp.float32)
        + b_ref[...])


def kernel(inputs, table, W, b):
    idx2d = jnp.transpose(inputs).reshape(_B * 2, _HALF)
    idx_flat = jnp.pad(idx2d, ((0, 0), (0, _IDXP - _HALF))).reshape(-1)
    mesh = plsc.VectorSubcoreMesh(core_axis_name="c", subcore_axis_name="s")
    pooled = pl.kernel(
        _sc_pool_body,
        mesh=mesh,
        out_type=jax.ShapeDtypeStruct((_B, _EMB), jnp.float32),
        scratch_types=[
            pltpu.VMEM((2 * _EPW * _IDXP,), jnp.int32),
        ] + [pltpu.VMEM((2 * _HALF, _EMB), jnp.float32)] * _NBUF + [
            pltpu.VMEM((_OUT_TILE, _EMB), jnp.float32),
        ] + [pltpu.SemaphoreType.DMA] * _NBUF,
    )(idx_flat, table)
    out = pl.pallas_call(
        _head_body,
        grid=(4,),
        in_specs=[
            pl.BlockSpec((_B // 4, _EMB), lambda i: (i, 0)),
            pl.BlockSpec((_EMB, _LBL), lambda i: (0, 0)),
            pl.BlockSpec((1, _LBL), lambda i: (0, 0)),
        ],
        out_specs=pl.BlockSpec((_B // 4, _LBL), lambda i: (i, 0)),
        out_shape=jax.ShapeDtypeStruct((_B, _LBL), jnp.float32),
    )(pooled, W, jnp.reshape(b, (1, _LBL)))
    return out


# speedup vs baseline: 15.4506x; 1.0037x over previous
import jax
import jax.numpy as jnp
from jax import lax
from jax.experimental import pallas as pl
from jax.experimental.pallas import tpu as pltpu
from jax.experimental.pallas import tpu_sc as plsc

_NC = 2
_NS = 16
_NW = _NC * _NS
_L = 200
_B = 4096
_EMB = 128
_LBL = 100
_EPW = _B // _NW
_HALF = _L // 2
_VEC = 16
_KV = _EMB // _VEC
_NBUF = 4
_IDXP = 104
_OUT_TILE = 8
_SCALE = 1.0 / _L


def _sc_pool_body(idx_hbm, table_hbm, out_hbm, idx_v,
                  rows0, rows1, rows2, rows3, outst,
                  sem0, sem1, sem2, sem3):
    bufs = (rows0, rows1, rows2, rows3)
    sems = (sem0, sem1, sem2, sem3)
    wid = lax.axis_index("s") * _NC + lax.axis_index("c")
    base = pl.multiple_of(wid * _EPW, _EPW)
    nwords = 2 * _EPW * _IDXP
    pltpu.sync_copy(
        idx_hbm.at[pl.ds(pl.multiple_of(wid * nwords, 8), nwords)], idx_v)

    def idx_ref(row):
        return idx_v.at[pl.ds(pl.multiple_of(row * _IDXP, 8), _HALF)]

    def issue(e, buf, sem):
        row = e * 2
        pltpu.async_copy(table_hbm.at[idx_ref(row)],
                         buf.at[pl.ds(0, _HALF)], sem)
        pltpu.async_copy(table_hbm.at[idx_ref(row + 1)],
                         buf.at[pl.ds(_HALF, _HALF)], sem)

    def drain(buf, sem):
        for r in range(2):
            pltpu.make_async_copy(table_hbm.at[idx_ref(0)],
                                  buf.at[pl.ds(0, _HALF)], sem).wait()

    def consume(e, buf):
        zeros = tuple(jnp.zeros((_VEC,), jnp.float32) for _ in range(_KV))

        @plsc.parallel_loop(0, _HALF, unroll=2, carry=zeros)
        def acc(l, acc):
            acc = list(acc)
            for h in range(2):
                for k in range(_KV):
                    acc[k] = acc[k] + buf[l + _HALF * h,
                                          pl.ds(k * _VEC, _VEC)]
            return tuple(acc)
        m = lax.rem(e, _OUT_TILE)
        for k in range(_KV):
            outst[m, pl.ds(k * _VEC, _VEC)] = acc[k] * _SCALE

        @pl.when(m == _OUT_TILE - 1)
        def _flush():
            start = pl.multiple_of(base + e - (_OUT_TILE - 1), _OUT_TILE)
            pltpu.sync_copy(outst, out_hbm.at[pl.ds(start, _OUT_TILE)])

    for q in range(_NBUF):
        issue(q, bufs[q], sems[q])

    def step(g, carry):
        e0 = _NBUF * g
        for q in range(_NBUF):
            e = e0 + q
            drain(bufs[q], sems[q])
            consume(e, bufs[q])

            @pl.when(e + _NBUF < _EPW)
            def _():
                issue(e + _NBUF, bufs[q], sems[q])
        return carry

    lax.fori_loop(0, _EPW // _NBUF, step, jnp.int32(0))


def _head_body(z_ref, w_ref, b_ref, o_ref):
    o_ref[...] = jax.nn.sigmoid(
        jnp.dot(z_ref[...], w_ref[...], preferred_element_type=jnp.float32)
        + b_ref[...])


def kernel(inputs, table, W, b):
    idx2d = jnp.transpose(inputs).reshape(_B * 2, _HALF)
    idx_flat = jnp.pad(idx2d, ((0, 0), (0, _IDXP - _HALF))).reshape(-1)
    mesh = plsc.VectorSubcoreMesh(core_axis_name="c", subcore_axis_name="s")
    pooled = pl.kernel(
        _sc_pool_body,
        mesh=mesh,
        out_type=jax.ShapeDtypeStruct((_B, _EMB), jnp.float32),
        scratch_types=[
            pltpu.VMEM((2 * _EPW * _IDXP,), jnp.int32),
        ] + [pltpu.VMEM((2 * _HALF, _EMB), jnp.float32)] * _NBUF + [
            pltpu.VMEM((_OUT_TILE, _EMB), jnp.float32),
        ] + [pltpu.SemaphoreType.DMA] * _NBUF,
    )(idx_flat, table)
    out = pl.pallas_call(
        _head_body,
        grid=(4,),
        in_specs=[
            pl.BlockSpec((_B // 4, _EMB), lambda i: (i, 0)),
            pl.BlockSpec((_EMB, _LBL), lambda i: (0, 0)),
            pl.BlockSpec((1, _LBL), lambda i: (0, 0)),
        ],
        out_specs=pl.BlockSpec((_B // 4, _LBL), lambda i: (i, 0)),
        out_shape=jax.ShapeDtypeStruct((_B, _LBL), jnp.float32),
    )(pooled, W, jnp.reshape(b, (1, _LBL)))
    return out


# speedup vs baseline: 15.7168x; 1.0172x over previous
import jax
import jax.numpy as jnp
from jax import lax
from jax.experimental import pallas as pl
from jax.experimental.pallas import tpu as pltpu
from jax.experimental.pallas import tpu_sc as plsc

_NC = 2
_NS = 16
_NW = _NC * _NS
_L = 200
_B = 4096
_EMB = 128
_LBL = 100
_EPW = _B // _NW
_HALF = _L // 2
_VEC = 16
_KV = _EMB // _VEC
_NBUF = 4
_G0 = 96
_G1 = _L - _G0
_OUT_TILE = 8
_SCALE = 1.0 / _L


def _sc_pool_body(idx_hbm, table_hbm, out_hbm, idx_v,
                  rows0, rows1, rows2, rows3, outst,
                  sem0, sem1, sem2, sem3):
    bufs = (rows0, rows1, rows2, rows3)
    sems = (sem0, sem1, sem2, sem3)
    wid = lax.axis_index("s") * _NC + lax.axis_index("c")
    base = pl.multiple_of(wid * _EPW, _EPW)
    nwords = _EPW * _L
    pltpu.sync_copy(
        idx_hbm.at[pl.ds(pl.multiple_of(wid * nwords, 8), nwords)], idx_v)

    def issue(e, buf, sem):
        off = pl.multiple_of(e * _L, 8)
        pltpu.async_copy(table_hbm.at[idx_v.at[pl.ds(off, _G0)]],
                         buf.at[pl.ds(0, _G0)], sem)
        off1 = pl.multiple_of(e * _L + _G0, 8)
        pltpu.async_copy(table_hbm.at[idx_v.at[pl.ds(off1, _G1)]],
                         buf.at[pl.ds(_G0, _G1)], sem)

    def drain(buf, sem):
        pltpu.make_async_copy(table_hbm.at[idx_v.at[pl.ds(0, _G0)]],
                              buf.at[pl.ds(0, _G0)], sem).wait()
        pltpu.make_async_copy(table_hbm.at[idx_v.at[pl.ds(0, _G1)]],
                              buf.at[pl.ds(0, _G1)], sem).wait()

    def consume(e, buf):
        zeros = tuple(jnp.zeros((_VEC,), jnp.float32) for _ in range(_KV))

        @plsc.parallel_loop(0, _HALF, unroll=2, carry=zeros)
        def acc(l, acc):
            acc = list(acc)
            for h in range(2):
                for k in range(_KV):
                    acc[k] = acc[k] + buf[l + _HALF * h,
                                          pl.ds(k * _VEC, _VEC)]
            return tuple(acc)
        m = lax.rem(e, _OUT_TILE)
        for k in range(_KV):
            outst[m, pl.ds(k * _VEC, _VEC)] = acc[k] * _SCALE

        @pl.when(m == _OUT_TILE - 1)
        def _flush():
            start = pl.multiple_of(base + e - (_OUT_TILE - 1), _OUT_TILE)
            pltpu.sync_copy(outst, out_hbm.at[pl.ds(start, _OUT_TILE)])

    for q in range(_NBUF):
        issue(q, bufs[q], sems[q])

    def step(g, carry):
        e0 = _NBUF * g
        for q in range(_NBUF):
            e = e0 + q
            drain(bufs[q], sems[q])
            consume(e, bufs[q])

            @pl.when(e + _NBUF < _EPW)
            def _():
                issue(e + _NBUF, bufs[q], sems[q])
        return carry

    lax.fori_loop(0, _EPW // _NBUF, step, jnp.int32(0))


def _head_body(z_ref, w_ref, b_ref, o_ref):
    o_ref[...] = jax.nn.sigmoid(
        jnp.dot(z_ref[...], w_ref[...], preferred_element_type=jnp.float32)
        + b_ref[...])


def kernel(inputs, table, W, b):
    idx_flat = jnp.transpose(inputs).reshape(-1)
    mesh = plsc.VectorSubcoreMesh(core_axis_name="c", subcore_axis_name="s")
    pooled = pl.kernel(
        _sc_pool_body,
        mesh=mesh,
        out_type=jax.ShapeDtypeStruct((_B, _EMB), jnp.float32),
        scratch_types=[
            pltpu.VMEM((_EPW * _L,), jnp.int32),
        ] + [pltpu.VMEM((2 * _HALF, _EMB), jnp.float32)] * _NBUF + [
            pltpu.VMEM((_OUT_TILE, _EMB), jnp.float32),
        ] + [pltpu.SemaphoreType.DMA] * _NBUF,
    )(idx_flat, table)
    out = pl.pallas_call(
        _head_body,
        grid=(4,),
        in_specs=[
            pl.BlockSpec((_B // 4, _EMB), lambda i: (i, 0)),
            pl.BlockSpec((_EMB, _LBL), lambda i: (0, 0)),
            pl.BlockSpec((1, _LBL), lambda i: (0, 0)),
        ],
        out_specs=pl.BlockSpec((_B // 4, _LBL), lambda i: (i, 0)),
        out_shape=jax.ShapeDtypeStruct((_B, _LBL), jnp.float32),
    )(pooled, W, jnp.reshape(b, (1, _LBL)))
    return out


# speedup vs baseline: 15.7625x; 1.0029x over previous
import jax
import jax.numpy as jnp
from jax import lax
from jax.experimental import pallas as pl
from jax.experimental.pallas import tpu as pltpu
from jax.experimental.pallas import tpu_sc as plsc

_NC = 2
_NS = 16
_NW = _NC * _NS
_L = 200
_B = 4096
_EMB = 128
_LBL = 100
_EPW = _B // _NW
_HALF = _L // 2
_VEC = 16
_KV = _EMB // _VEC
_NBUF = 4
_G0 = 96
_G1 = _L - _G0
_OUT_TILE = 8
_SCALE = 1.0 / _L


def _sc_pool_body(idx_hbm, table_hbm, out_hbm, idx_v,
                  rows0, rows1, rows2, rows3, outst,
                  sem0, sem1, sem2, sem3, out_sem):
    bufs = (rows0, rows1, rows2, rows3)
    sems = (sem0, sem1, sem2, sem3)
    wid = lax.axis_index("s") * _NC + lax.axis_index("c")
    base = pl.multiple_of(wid * _EPW, _EPW)
    nwords = _EPW * _L
    pltpu.sync_copy(
        idx_hbm.at[pl.ds(pl.multiple_of(wid * nwords, 8), nwords)], idx_v)

    def issue(e, buf, sem):
        off = pl.multiple_of(e * _L, 8)
        pltpu.async_copy(table_hbm.at[idx_v.at[pl.ds(off, _G0)]],
                         buf.at[pl.ds(0, _G0)], sem)
        off1 = pl.multiple_of(e * _L + _G0, 8)
        pltpu.async_copy(table_hbm.at[idx_v.at[pl.ds(off1, _G1)]],
                         buf.at[pl.ds(_G0, _G1)], sem)

    def drain(buf, sem):
        pltpu.make_async_copy(table_hbm.at[idx_v.at[pl.ds(0, _G0)]],
                              buf.at[pl.ds(0, _G0)], sem).wait()
        pltpu.make_async_copy(table_hbm.at[idx_v.at[pl.ds(0, _G1)]],
                              buf.at[pl.ds(0, _G1)], sem).wait()

    def consume(e, buf):
        zeros = tuple(jnp.zeros((_VEC,), jnp.float32) for _ in range(_KV))

        @plsc.parallel_loop(0, _HALF, unroll=2, carry=zeros)
        def acc(l, acc):
            acc = list(acc)
            for h in range(2):
                for k in range(_KV):
                    acc[k] = acc[k] + buf[l + _HALF * h,
                                          pl.ds(k * _VEC, _VEC)]
            return tuple(acc)
        m = lax.rem(e, _OUT_TILE)

        @pl.when(jnp.logical_and(m == 0, e >= _OUT_TILE))
        def _drain_prev_flush():
            pltpu.make_async_copy(out_hbm.at[pl.ds(0, _OUT_TILE)], outst,
                                  out_sem).wait()

        for k in range(_KV):
            outst[m, pl.ds(k * _VEC, _VEC)] = acc[k] * _SCALE

        @pl.when(m == _OUT_TILE - 1)
        def _flush():
            start = pl.multiple_of(base + e - (_OUT_TILE - 1), _OUT_TILE)
            pltpu.async_copy(outst, out_hbm.at[pl.ds(start, _OUT_TILE)],
                             out_sem)

    for q in range(_NBUF):
        issue(q, bufs[q], sems[q])

    def step(g, carry):
        e0 = _NBUF * g
        for q in range(_NBUF):
            e = e0 + q
            drain(bufs[q], sems[q])
            consume(e, bufs[q])

            @pl.when(e + _NBUF < _EPW)
            def _():
                issue(e + _NBUF, bufs[q], sems[q])
        return carry

    lax.fori_loop(0, _EPW // _NBUF, step, jnp.int32(0))
    pltpu.make_async_copy(out_hbm.at[pl.ds(0, _OUT_TILE)], outst,
                          out_sem).wait()


def _head_body(z_ref, w_ref, b_ref, o_ref):
    o_ref[...] = jax.nn.sigmoid(
        jnp.dot(z_ref[...], w_ref[...], preferred_element_type=jnp.float32)
        + b_ref[...])


def kernel(inputs, table, W, b):
    idx_flat = jnp.transpose(inputs).reshape(-1)
    mesh = plsc.VectorSubcoreMesh(core_axis_name="c", subcore_axis_name="s")
    pooled = pl.kernel(
        _sc_pool_body,
        mesh=mesh,
        out_type=jax.ShapeDtypeStruct((_B, _EMB), jnp.float32),
        scratch_types=[
            pltpu.VMEM((_EPW * _L,), jnp.int32),
        ] + [pltpu.VMEM((2 * _HALF, _EMB), jnp.float32)] * _NBUF + [
            pltpu.VMEM((_OUT_TILE, _EMB), jnp.float32),
        ] + [pltpu.SemaphoreType.DMA] * (_NBUF + 1),
    )(idx_flat, table)
    out = pl.pallas_call(
        _head_body,
        grid=(4,),
        in_specs=[
            pl.BlockSpec((_B // 4, _EMB), lambda i: (i, 0)),
            pl.BlockSpec((_EMB, _LBL), lambda i: (0, 0)),
            pl.BlockSpec((1, _LBL), lambda i: (0, 0)),
        ],
        out_specs=pl.BlockSpec((_B // 4, _LBL), lambda i: (i, 0)),
        out_shape=jax.ShapeDtypeStruct((_B, _LBL), jnp.float32),
    )(pooled, W, jnp.reshape(b, (1, _LBL)))
    return out


# speedup vs baseline: 15.7894x; 1.0017x over previous
import jax
import jax.numpy as jnp
from jax import lax
from jax.experimental import pallas as pl
from jax.experimental.pallas import tpu as pltpu
from jax.experimental.pallas import tpu_sc as plsc

_NC = 2
_NS = 16
_NW = _NC * _NS
_L = 200
_B = 4096
_EMB = 128
_LBL = 100
_EPW = _B // _NW
_HALF = _L // 2
_VEC = 16
_KV = _EMB // _VEC
_NBUF = 4
_G0 = 96
_G1 = _L - _G0
_OUT_TILE = 8
_SCALE = 1.0 / _L


def _sc_pool_body(idx_hbm, table_hbm, out_hbm, idx_v,
                  rows0, rows1, rows2, rows3, outst,
                  sem0, sem1, sem2, sem3, out_sem):
    bufs = (rows0, rows1, rows2, rows3)
    sems = (sem0, sem1, sem2, sem3)
    wid = lax.axis_index("s") * _NC + lax.axis_index("c")
    base = pl.multiple_of(wid * _EPW, _EPW)
    nwords = _EPW * _L
    pltpu.sync_copy(
        idx_hbm.at[pl.ds(pl.multiple_of(wid * nwords, 8), nwords)], idx_v)

    def issue(e, buf, sem):
        off = pl.multiple_of(e * _L, 8)
        pltpu.async_copy(table_hbm.at[idx_v.at[pl.ds(off, _G0)]],
                         buf.at[pl.ds(0, _G0)], sem)
        off1 = pl.multiple_of(e * _L + _G0, 8)
        pltpu.async_copy(table_hbm.at[idx_v.at[pl.ds(off1, _G1)]],
                         buf.at[pl.ds(_G0, _G1)], sem)

    def drain(buf, sem):
        pltpu.make_async_copy(table_hbm.at[idx_v.at[pl.ds(0, _G0)]],
                              buf.at[pl.ds(0, _G0)], sem).wait()
        pltpu.make_async_copy(table_hbm.at[idx_v.at[pl.ds(0, _G1)]],
                              buf.at[pl.ds(0, _G1)], sem).wait()

    def consume(e, buf):
        zeros = tuple(jnp.zeros((_VEC,), jnp.float32) for _ in range(_KV))

        @plsc.parallel_loop(0, _HALF, unroll=2, carry=zeros)
        def acc(l, acc):
            acc = list(acc)
            for h in range(2):
                for k in range(_KV):
                    acc[k] = acc[k] + buf[l + _HALF * h,
                                          pl.ds(k * _VEC, _VEC)]
            return tuple(acc)
        m = lax.rem(e, _OUT_TILE)

        @pl.when(jnp.logical_and(m == 0, e >= _OUT_TILE))
        def _drain_prev_flush():
            pltpu.make_async_copy(out_hbm.at[pl.ds(0, _OUT_TILE)], outst,
                                  out_sem).wait()

        for k in range(_KV):
            outst[m, pl.ds(k * _VEC, _VEC)] = acc[k] * _SCALE

        @pl.when(m == _OUT_TILE - 1)
        def _flush():
            start = pl.multiple_of(base + e - (_OUT_TILE - 1), _OUT_TILE)
            pltpu.async_copy(outst, out_hbm.at[pl.ds(start, _OUT_TILE)],
                             out_sem)

    for q in range(_NBUF):
        issue(q, bufs[q], sems[q])

    def step(g, carry):
        e0 = _NBUF * g
        for q in range(_NBUF):
            e = e0 + q
            drain(bufs[q], sems[q])
            consume(e, bufs[q])

            @pl.when(e + _NBUF < _EPW)
            def _():
                issue(e + _NBUF, bufs[q], sems[q])
        return carry

    lax.fori_loop(0, _EPW // _NBUF, step, jnp.int32(0))
    pltpu.make_async_copy(out_hbm.at[pl.ds(0, _OUT_TILE)], outst,
                          out_sem).wait()


def _head_body(z_ref, w_ref, b_ref, o_ref):
    o_ref[...] = jax.nn.sigmoid(
        jnp.dot(z_ref[...], w_ref[...], preferred_element_type=jnp.float32)
        + b_ref[...])


def kernel(inputs, table, W, b):
    idx_flat = jnp.transpose(inputs).reshape(-1)
    mesh = plsc.VectorSubcoreMesh(core_axis_name="c", subcore_axis_name="s")
    pooled = pl.kernel(
        _sc_pool_body,
        mesh=mesh,
        out_type=jax.ShapeDtypeStruct((_B, _EMB), jnp.float32),
        scratch_types=[
            pltpu.VMEM((_EPW * _L,), jnp.int32),
        ] + [pltpu.VMEM((2 * _HALF, _EMB), jnp.float32)] * _NBUF + [
            pltpu.VMEM((_OUT_TILE, _EMB), jnp.float32),
        ] + [pltpu.SemaphoreType.DMA] * (_NBUF + 1),
    )(idx_flat, table)
    out = pl.pallas_call(
        _head_body,
        out_shape=jax.ShapeDtypeStruct((_B, _LBL), jnp.float32),
    )(pooled, W, jnp.reshape(b, (1, _LBL)))
    return out
